# baseline (speedup 1.0000x reference)
import functools

import jax
import jax.numpy as jnp
import numpy as np
from jax.experimental import pallas as pl

_C = 384
_K = 20
_NP = 96
_H = 14
_W = 14
_ROWS_PER_BLOCK = 4
_U = 224


def _perm76():
    return jax.random.permutation(
        jax.random.key(1), _H * _W - _K)[: _NP - _K]


def _body(xp_ref, wt_ref, dwb_ref, wc_ref, pwb_ref, perm_ref, x_ref, y_ref):
    R = _ROWS_PER_BLOCK
    xpb = xp_ref[...]
    acc = None
    for dy in range(3):
        for dx in range(3):
            o = dy * 16 + dx
            term = xpb[:, o:o + _U, :].astype(jnp.float32) * wt_ref[dy, dx][None, None, :]
            acc = term if acc is None else acc + term
    acc = acc + dwb_ref[...][0][None, None, :]
    tb = acc.astype(jnp.bfloat16)
    p = jax.lax.dot_general(tb.reshape(R * _U, _C), wc_ref[...],
                            (((1,), (0,)), ((), ())),
                            preferred_element_type=jnp.float32)
    p = p[:, 0].reshape(R, _U) + pwb_ref[...][0, 0]
    s = 1.0 / (1.0 + jnp.exp(-p))

    uu = jax.lax.broadcasted_iota(jnp.int32, (1, 1, _U), 2)
    jj = jax.lax.rem(uu, 16)
    ii = jax.lax.div(uu, 16)
    valid3 = jj < _W
    s = jnp.where(valid3[:, 0, :], s, -1.0)

    sa = s[:, :, None]
    sb = s[:, None, :]
    aidx = jax.lax.broadcasted_iota(jnp.int32, (1, _U, 1), 1)
    less = uu < aidx
    m = (sb > sa) | ((sb == sa) & less)
    rank = jnp.sum(m.astype(jnp.float32), axis=2)

    unsel = (rank >= float(_K)) & valid3[0]
    l_mat = less & unsel[:, None, :]
    ur = jnp.sum(l_mat.astype(jnp.float32), axis=2)

    clipx = jnp.clip(jj, 1, _W - 1).astype(jnp.float32)
    clipy = jnp.clip(ii, 1, _H - 1).astype(jnp.float32)

    karr = jax.lax.broadcasted_iota(jnp.int32, (1, _K, 1), 1).astype(jnp.float32)
    oh20 = (rank[:, None, :] == karr).astype(jnp.float32)
    x_sel = jnp.sum(oh20 * clipx, axis=2)
    y_sel = jnp.sum(oh20 * clipy, axis=2)

    perm3 = perm_ref[...][:, 0:1][None, :, 0:1]
    oh76 = ((ur[:, None, :] == perm3) & unsel[:, None, :]).astype(jnp.float32)
    x_rem = jnp.sum(oh76 * clipx, axis=2)
    y_rem = jnp.sum(oh76 * clipy, axis=2)

    x_ref[...] = jnp.concatenate((x_sel, x_rem), axis=1).astype(jnp.int32)[None]
    y_ref[...] = jnp.concatenate((y_sel, y_rem), axis=1).astype(jnp.int32)[None]


def kernel(feature, dw_w, dw_b, pw_w, pw_b):
    b, n, c, h, w = feature.shape
    xb = feature[0].astype(jnp.bfloat16)
    xpad = jnp.zeros((n, 17, 16, c), jnp.bfloat16).at[:, 1:15, 1:15, :].set(
        jnp.transpose(xb, (0, 2, 3, 1))).reshape(n, 272, c)
    wt = jnp.transpose(dw_w[:, 0], (1, 2, 0))
    dwb = jnp.broadcast_to(dw_b[None, :], (8, c))
    wcol = jnp.zeros((c, 128), jnp.float32).at[:, 0].set(pw_w[0, :, 0, 0])
    pwb = jnp.broadcast_to(pw_b[:, None], (8, 128))
    perm = jnp.broadcast_to(
        _perm76().astype(jnp.float32)[:, None], (_NP - _K, 128))

    R = _ROWS_PER_BLOCK
    f = pl.pallas_call(
        _body,
        grid=(n // R,),
        in_specs=[pl.BlockSpec((R, 272, c), lambda i: (i, 0, 0)),
                  pl.BlockSpec((3, 3, c), lambda i: (0, 0, 0)),
                  pl.BlockSpec((8, c), lambda i: (0, 0)),
                  pl.BlockSpec((c, 128), lambda i: (0, 0)),
                  pl.BlockSpec((8, 128), lambda i: (0, 0)),
                  pl.BlockSpec((_NP - _K, 128), lambda i: (0, 0))],
        out_specs=[pl.BlockSpec((1, R, _NP), lambda i: (i, 0, 0)),
                   pl.BlockSpec((1, R, _NP), lambda i: (i, 0, 0))],
        out_shape=[jax.ShapeDtypeStruct((n // R, R, _NP), jnp.int32),
                   jax.ShapeDtypeStruct((n // R, R, _NP), jnp.int32)],
    )
    x, y = f(xpad, wt, dwb, wcol, pwb, perm)
    return (x.reshape(n, _NP), y.reshape(n, _NP))
